# baseline (speedup 1.0000x reference)
import functools

import jax
import jax.numpy as jnp
from jax.experimental import pallas as pl


def _quant_block(bits_ref, f_ref, o_ref):
    f = f_ref[...]
    bits = bits_ref[...]
    bits = jnp.clip(bits, 1, 8)
    lm1 = (jnp.exp2(bits.astype(jnp.float32)) - 1.0)[:, None, :, :]

    f_min = jnp.min(f, axis=1, keepdims=True)
    f_max = jnp.max(f, axis=1, keepdims=True)
    rng = f_max - f_min
    valid = rng > 1e-8
    scale_up = jnp.where(valid, lm1 / jnp.where(valid, rng, 1.0), 0.0)
    scale_dn = rng / lm1
    q = jnp.round((f - f_min) * scale_up)
    o_ref[...] = q * scale_dn + f_min


@functools.partial(jax.jit, static_argnames=("hb",))
def _run(features, bits_i32, hb=56):
    b, c, h, w = features.shape
    hw = h * w
    rows = hw // 128
    f4 = features.reshape(b, c, rows, 128)
    bits4 = bits_i32.reshape(b, rows, 128)
    grid = (b, rows // hb)
    out = pl.pallas_call(
        _quant_block,
        grid=grid,
        in_specs=[
            pl.BlockSpec((1, hb, 128), lambda i, j: (i, j, 0)),
            pl.BlockSpec((1, c, hb, 128), lambda i, j: (i, 0, j, 0)),
        ],
        out_specs=pl.BlockSpec((1, c, hb, 128), lambda i, j: (i, 0, j, 0)),
        out_shape=jax.ShapeDtypeStruct(f4.shape, f4.dtype),
    )(bits4, f4)
    return out.reshape(b, c, h, w)


def kernel(features, bit_allocation):
    return _run(features, bit_allocation.astype(jnp.int32))


# speedup vs baseline: 4.1736x; 4.1736x over previous
import functools

import jax
import jax.numpy as jnp
from jax.experimental import pallas as pl


def _quant_block(bits_ref, f_ref, o_ref):
    bits = bits_ref[0]
    bits = jnp.clip(bits, 1, 8)
    lm1 = jnp.exp2(bits.astype(jnp.float32)) - 1.0

    c = f_ref.shape[1]
    f_min = f_ref[0, 0]
    f_max = f_min
    for k in range(1, c):
        fk = f_ref[0, k]
        f_min = jnp.minimum(f_min, fk)
        f_max = jnp.maximum(f_max, fk)
    rng = f_max - f_min
    valid = rng > 1e-8
    scale_up = jnp.where(valid, lm1 / jnp.where(valid, rng, 1.0), 0.0)
    scale_dn = rng / lm1
    for k in range(c):
        fk = f_ref[0, k]
        o_ref[0, k] = jnp.round((fk - f_min) * scale_up) * scale_dn + f_min


@functools.partial(jax.jit, static_argnames=("hb",))
def _run(features, bits_i32, hb=32):
    b, c, h, w = features.shape
    grid = (b, h // hb)
    return pl.pallas_call(
        _quant_block,
        grid=grid,
        in_specs=[
            pl.BlockSpec((1, hb, w), lambda i, j: (i, j, 0)),
            pl.BlockSpec((1, c, hb, w), lambda i, j: (i, 0, j, 0)),
        ],
        out_specs=pl.BlockSpec((1, c, hb, w), lambda i, j: (i, 0, j, 0)),
        out_shape=jax.ShapeDtypeStruct(features.shape, features.dtype),
    )(bits_i32, features)


def kernel(features, bit_allocation):
    return _run(features, bit_allocation.astype(jnp.int32))


# speedup vs baseline: 4.5106x; 1.0808x over previous
import functools

import jax
import jax.numpy as jnp
from jax.experimental import pallas as pl


def _quant_block(bits_ref, f_ref, o_ref):
    bits = bits_ref[0]
    bits = jnp.clip(bits, 1, 8)
    lm1 = jnp.exp2(bits.astype(jnp.float32)) - 1.0

    c = f_ref.shape[1]
    f_min = f_ref[0, 0]
    f_max = f_min
    for k in range(1, c):
        fk = f_ref[0, k]
        f_min = jnp.minimum(f_min, fk)
        f_max = jnp.maximum(f_max, fk)
    rng = f_max - f_min
    valid = rng > 1e-8
    scale_up = jnp.where(valid, lm1 / jnp.where(valid, rng, 1.0), 0.0)
    scale_dn = rng / lm1
    for k in range(c):
        fk = f_ref[0, k]
        o_ref[0, k] = jnp.round((fk - f_min) * scale_up) * scale_dn + f_min


@functools.partial(jax.jit, static_argnames=("hb",))
def _run(features, bits_i32, hb=112):
    b, c, h, w = features.shape
    grid = (b, h // hb)
    return pl.pallas_call(
        _quant_block,
        grid=grid,
        in_specs=[
            pl.BlockSpec((1, hb, w), lambda i, j: (i, j, 0)),
            pl.BlockSpec((1, c, hb, w), lambda i, j: (i, 0, j, 0)),
        ],
        out_specs=pl.BlockSpec((1, c, hb, w), lambda i, j: (i, 0, j, 0)),
        out_shape=jax.ShapeDtypeStruct(features.shape, features.dtype),
    )(bits_i32, features)


def kernel(features, bit_allocation):
    return _run(features, bit_allocation.astype(jnp.int32))
